# baseline (speedup 1.0000x reference)
import functools

import jax
import jax.numpy as jnp
from jax import lax
from jax.experimental import pallas as pl
from jax.experimental.pallas import tpu as pltpu
from jax.experimental.pallas import tpu_sc as plsc

B, N, D, KNN = 2, 4096, 128, 16
PPAD = 16
BQ = 512
BQ_L = 128
NBLK = N // BQ
ROWS = B * N * KNN


def _prep_body(x_ref, pos_ref, wq_ref, wk_ref, wv_ref, q_ref, kv_ref, p_ref):
    x = x_ref[...]
    q_ref[...] = jnp.dot(x, wq_ref[...], preferred_element_type=jnp.float32)
    xk = jnp.dot(x, wk_ref[...], preferred_element_type=jnp.float32)
    xv = jnp.dot(x, wv_ref[...], preferred_element_type=jnp.float32)
    kb = lax.bitcast_convert_type(xk.astype(jnp.bfloat16), jnp.uint16)
    vb = lax.bitcast_convert_type(xv.astype(jnp.bfloat16), jnp.uint16)
    kv_ref[...] = lax.shift_left(kb.astype(jnp.int32), 16) | vb.astype(jnp.int32)
    pos = pos_ref[...]
    p_ref[...] = jnp.concatenate(
        [pos, jnp.zeros((pos.shape[0], D - 3), jnp.float32)], axis=-1)


def _prep(xf, posf, Wq, Wk, Wv):
    blk = 512
    grid = (B * N // blk,)
    return pl.pallas_call(
        _prep_body,
        grid=grid,
        in_specs=[
            pl.BlockSpec((blk, D), lambda i: (i, 0)),
            pl.BlockSpec((blk, 3), lambda i: (i, 0)),
            pl.BlockSpec((D, D), lambda i: (0, 0)),
            pl.BlockSpec((D, D), lambda i: (0, 0)),
            pl.BlockSpec((D, D), lambda i: (0, 0)),
        ],
        out_specs=[
            pl.BlockSpec((blk, D), lambda i: (i, 0)),
            pl.BlockSpec((blk, D), lambda i: (i, 0)),
            pl.BlockSpec((blk, D), lambda i: (i, 0)),
        ],
        out_shape=[
            jax.ShapeDtypeStruct((B * N, D), jnp.float32),
            jax.ShapeDtypeStruct((B * N, D), jnp.int32),
            jax.ShapeDtypeStruct((B * N, D), jnp.float32),
        ],
    )(xf, posf, Wq, Wk, Wv)


def _topk_body(posq_ref, posT_ref, out_ref, *, b):
    qp = posq_ref[0]
    pT = posT_ref[0]
    qx, qy, qz = qp[:, 0], qp[:, 1], qp[:, 2]
    qs2 = ((qx * qx + qy * qy) + qz * qz)[:, None]
    px, py, pz = pT[0, :], pT[1, :], pT[2, :]
    s2 = ((px * px + py * py) + pz * pz)[None, :]
    qp8 = jnp.concatenate([qp, jnp.zeros((BQ, 5), jnp.float32)], axis=1)
    cross = jnp.dot(qp8.astype(jnp.bfloat16), pT.astype(jnp.bfloat16),
                    preferred_element_type=jnp.float32)
    d = (qs2 + s2) - 2.0 * cross

    NP = N // BQ_L
    INF = jnp.float32(jnp.inf)
    pv = [jnp.full((BQ, BQ_L), INF, jnp.float32) for _ in range(4)]
    pg = [jnp.zeros((BQ, BQ_L), jnp.int32) for _ in range(4)]
    lane = lax.broadcasted_iota(jnp.int32, (BQ, BQ_L), 1)
    for p in range(NP):
        tv = d[:, p * BQ_L:(p + 1) * BQ_L]
        tg = lane + p * BQ_L
        for s in range(4):
            c = tv < pv[s]
            nv = jnp.where(c, tv, pv[s])
            ng = jnp.where(c, tg, pg[s])
            tv = jnp.where(c, pv[s], tv)
            tg = jnp.where(c, pg[s], tg)
            pv[s] = nv
            pg[s] = ng

    PV = jnp.concatenate(pv, axis=1)
    PG = jnp.concatenate(pg, axis=1)
    BIG = jnp.int32(1 << 30)
    cols = []
    m = None
    gi = None
    for _ in range(KNN):
        m = jnp.min(PV, axis=1, keepdims=True)
        eq = PV <= m
        gi = jnp.min(jnp.where(eq, PG, BIG), axis=1, keepdims=True)
        PV = jnp.where(eq & (PG == gi), INF, PV)
        cols.append(gi)
    pooled = jnp.concatenate(cols, axis=1)

    bad = (pv[3] < m) | ((pv[3] == m) & (pg[3] < gi))
    any_bad = jnp.sum(bad.astype(jnp.int32)) > 0

    def _full_extract(_):
        iota = lax.broadcasted_iota(jnp.int32, (BQ, N), 1)
        dd = d
        cs = []
        for _ in range(KNN):
            mm = jnp.min(dd, axis=1, keepdims=True)
            eq2 = dd <= mm
            ix = jnp.min(jnp.where(eq2, iota, N), axis=1, keepdims=True)
            dd = jnp.where(iota == ix, INF, dd)
            cs.append(ix)
        return jnp.concatenate(cs, axis=1)

    out = lax.cond(any_bad, _full_extract, lambda _: pooled, 0)
    out_ref[0] = out + b * N


def _topk(pos_b, posT_b, b):
    grid = (NBLK,)
    return pl.pallas_call(
        functools.partial(_topk_body, b=b),
        grid=grid,
        in_specs=[
            pl.BlockSpec((1, BQ, 3), lambda i: (0, i, 0)),
            pl.BlockSpec((1, 8, N), lambda i: (0, 0, 0)),
        ],
        out_specs=pl.BlockSpec((1, BQ, KNN), lambda i: (0, i, 0)),
        out_shape=jax.ShapeDtypeStruct((1, N, KNN), jnp.int32),
    )(pos_b, posT_b)


_SC_CHUNK = 128


def _sc_gather(kv, p128, idx_flat):
    info = plsc.get_sparse_core_info()
    nw = info.num_cores * info.num_subcores
    nrows = idx_flat.shape[0]
    rows_per_w = nrows // nw
    n_chunks = rows_per_w // _SC_CHUNK
    mesh = plsc.VectorSubcoreMesh(core_axis_name="c", subcore_axis_name="s")

    @functools.partial(
        pl.kernel,
        out_type=[
            jax.ShapeDtypeStruct((nrows, D), jnp.int32),
            jax.ShapeDtypeStruct((nrows, D), jnp.float32),
        ],
        mesh=mesh,
        scratch_types=[
            pltpu.VMEM((_SC_CHUNK,), jnp.int32),
            pltpu.VMEM((_SC_CHUNK, D), jnp.int32),
            pltpu.VMEM((_SC_CHUNK, D), jnp.float32),
            pltpu.SemaphoreType.DMA,
            pltpu.SemaphoreType.DMA,
        ],
    )
    def k(kv_hbm, p_hbm, idx_hbm, gkv_hbm, gp_hbm, idx_v, rkv, rp, s1, s2):
        wid = lax.axis_index("s") * info.num_cores + lax.axis_index("c")
        base_w = wid * rows_per_w

        def body(c, carry):
            base = base_w + c * _SC_CHUNK
            pltpu.sync_copy(idx_hbm.at[pl.ds(base, _SC_CHUNK)], idx_v)
            cp1 = pltpu.async_copy(kv_hbm.at[idx_v], rkv, s1)
            cp2 = pltpu.async_copy(p_hbm.at[idx_v], rp, s2)
            cp1.wait()
            cp2.wait()
            pltpu.sync_copy(rkv, gkv_hbm.at[pl.ds(base, _SC_CHUNK)])
            pltpu.sync_copy(rp, gp_hbm.at[pl.ds(base, _SC_CHUNK)])
            return carry

        lax.fori_loop(0, n_chunks, body, 0)

    return k(kv, p128, idx_flat)


_ISQ = 1.0 / (128.0 ** 0.5)


def _attn_body(q_ref, x_ref, pq_ref, gkv_ref, gp_ref, p1_ref, pb1_ref,
               p2_ref, pb2_ref, a1_ref, ab1_ref, a2_ref, ab2_ref,
               wf_ref, bf_ref, out_ref):
    R = BQ * KNN
    w = gkv_ref[...]
    kf = lax.bitcast_convert_type(w & jnp.int32(-65536), jnp.float32)
    vf = lax.bitcast_convert_type(lax.shift_left(w, 16), jnp.float32)
    posn = gp_ref[:, :PPAD]
    pq = pq_ref[:, :PPAD]
    pq_rep = jnp.broadcast_to(pq[:, None, :], (BQ, KNN, PPAD)).reshape(R, PPAD)
    rel = pq_rep - posn
    pe = jnp.dot(rel, p1_ref[...], preferred_element_type=jnp.float32)
    pe = jnp.maximum(pe + pb1_ref[...][None, :], 0.0)
    pe = jnp.dot(pe, p2_ref[...], preferred_element_type=jnp.float32)
    pe = pe + pb2_ref[...][None, :]

    q = q_ref[...]
    q_rep = jnp.broadcast_to(q[:, None, :], (BQ, KNN, D)).reshape(R, D)
    h = q_rep - kf + pe
    a = jnp.dot(h, a1_ref[...], preferred_element_type=jnp.float32)
    a = jnp.maximum(a + ab1_ref[...][None, :], 0.0)
    a = jnp.dot(a, a2_ref[...], preferred_element_type=jnp.float32)
    a = (a + ab2_ref[...][None, :]) * _ISQ

    a3 = a.reshape(BQ, KNN, D)
    m = jnp.max(a3, axis=1, keepdims=True)
    e = jnp.exp(a3 - m)
    s = jnp.sum(e, axis=1, keepdims=True)
    p = e / s
    v3 = (vf + pe).reshape(BQ, KNN, D)
    agg = jnp.sum(p * v3, axis=1)
    out = jnp.dot(agg, wf_ref[...], preferred_element_type=jnp.float32)
    out_ref[...] = out + bf_ref[...][None, :] + x_ref[...]


def _attn(q, xf, pq, gkv, gp, P1p, pb1, P2, pb2, A1, ab1, A2, ab2, Wf, bf):
    grid = (q.shape[0] // BQ,)
    R = BQ * KNN
    full = lambda shape: pl.BlockSpec(shape, lambda i: tuple(0 for _ in shape))
    return pl.pallas_call(
        _attn_body,
        grid=grid,
        in_specs=[
            pl.BlockSpec((BQ, D), lambda i: (i, 0)),
            pl.BlockSpec((BQ, D), lambda i: (i, 0)),
            pl.BlockSpec((BQ, D), lambda i: (i, 0)),
            pl.BlockSpec((R, D), lambda i: (i, 0)),
            pl.BlockSpec((R, D), lambda i: (i, 0)),
            full((PPAD, D)), full((D,)), full((D, D)), full((D,)),
            full((D, D)), full((D,)), full((D, D)), full((D,)),
            full((D, D)), full((D,)),
        ],
        out_specs=pl.BlockSpec((BQ, D), lambda i: (i, 0)),
        out_shape=jax.ShapeDtypeStruct((q.shape[0], D), jnp.float32),
    )(q, xf, pq, gkv, gp, P1p, pb1, P2, pb2, A1, ab1, A2, ab2, Wf, bf)


def kernel(x, pos, Wq, Wk, Wv, P1, pb1, P2, pb2, A1, ab1, A2, ab2, Wf, bf):
    xf = x.reshape(B * N, D)
    posf = pos.reshape(B * N, 3)
    q, kv, p128 = _prep(xf, posf, Wq, Wk, Wv)

    posT = jnp.swapaxes(pos, 1, 2)
    posT8 = jnp.concatenate(
        [posT, jnp.zeros((B, 5, N), jnp.float32)], axis=1)

    P1p = jnp.concatenate([P1, jnp.zeros((PPAD - 3, D), jnp.float32)], axis=0)

    knn = [None] * B
    gs = [None] * B
    outs = [None] * B
    for b in range(B):
        knn[b] = _topk(pos[b:b + 1], posT8[b:b + 1], b)
        gs[b] = _sc_gather(kv, p128, knn[b].reshape(N * KNN))
    for b in range(B):
        sl = slice(b * N, (b + 1) * N)
        outs[b] = _attn(q[sl], xf[sl], p128[sl], gs[b][0], gs[b][1],
                        P1p, pb1, P2, pb2, A1, ab1, A2, ab2, Wf, bf)
    out = jnp.concatenate(outs, axis=0)
    return out.reshape(B, N, D)


# speedup vs baseline: 1.6614x; 1.6614x over previous
import functools

import jax
import jax.numpy as jnp
from jax import lax
from jax.experimental import pallas as pl
from jax.experimental.pallas import tpu as pltpu
from jax.experimental.pallas import tpu_sc as plsc

B, N, D, KNN = 2, 4096, 128, 16
PPAD = 16
BQ = 256
BQ_L = 128
NBLK = N // BQ
ROWS = B * N * KNN


def _prep_body(x_ref, pos_ref, wq_ref, wk_ref, wv_ref, q_ref, kv_ref, p_ref):
    x = x_ref[...]
    q_ref[...] = jnp.dot(x, wq_ref[...], preferred_element_type=jnp.float32)
    xk = jnp.dot(x, wk_ref[...], preferred_element_type=jnp.float32)
    xv = jnp.dot(x, wv_ref[...], preferred_element_type=jnp.float32)
    kb = lax.bitcast_convert_type(xk.astype(jnp.bfloat16), jnp.uint16)
    vb = lax.bitcast_convert_type(xv.astype(jnp.bfloat16), jnp.uint16)
    kv_ref[...] = lax.shift_left(kb.astype(jnp.int32), 16) | vb.astype(jnp.int32)
    pos = pos_ref[...]
    p_ref[...] = jnp.concatenate(
        [pos, jnp.zeros((pos.shape[0], D - 3), jnp.float32)], axis=-1)


def _prep(xf, posf, Wq, Wk, Wv):
    blk = 512
    grid = (B * N // blk,)
    return pl.pallas_call(
        _prep_body,
        grid=grid,
        in_specs=[
            pl.BlockSpec((blk, D), lambda i: (i, 0)),
            pl.BlockSpec((blk, 3), lambda i: (i, 0)),
            pl.BlockSpec((D, D), lambda i: (0, 0)),
            pl.BlockSpec((D, D), lambda i: (0, 0)),
            pl.BlockSpec((D, D), lambda i: (0, 0)),
        ],
        out_specs=[
            pl.BlockSpec((blk, D), lambda i: (i, 0)),
            pl.BlockSpec((blk, D), lambda i: (i, 0)),
            pl.BlockSpec((blk, D), lambda i: (i, 0)),
        ],
        out_shape=[
            jax.ShapeDtypeStruct((B * N, D), jnp.float32),
            jax.ShapeDtypeStruct((B * N, D), jnp.int32),
            jax.ShapeDtypeStruct((B * N, D), jnp.float32),
        ],
    )(xf, posf, Wq, Wk, Wv)


def _topk_body(posq_ref, posT_ref, out_ref, *, b):
    qp = posq_ref[0]
    pT = posT_ref[0]
    qx, qy, qz = qp[:, 0], qp[:, 1], qp[:, 2]
    qs2 = ((qx * qx + qy * qy) + qz * qz)[:, None]
    px, py, pz = pT[0, :], pT[1, :], pT[2, :]
    s2 = ((px * px + py * py) + pz * pz)[None, :]
    qp8 = jnp.concatenate([qp, jnp.zeros((BQ, 5), jnp.float32)], axis=1)
    cross = jnp.dot(qp8.astype(jnp.bfloat16), pT.astype(jnp.bfloat16),
                    preferred_element_type=jnp.float32)
    d = (qs2 + s2) - 2.0 * cross

    NP = N // BQ_L
    INF = jnp.float32(jnp.inf)
    pv = [jnp.full((BQ, BQ_L), INF, jnp.float32) for _ in range(4)]
    pg = [jnp.zeros((BQ, BQ_L), jnp.int32) for _ in range(4)]
    lane = lax.broadcasted_iota(jnp.int32, (BQ, BQ_L), 1)
    for p in range(NP):
        tv = d[:, p * BQ_L:(p + 1) * BQ_L]
        tg = lane + p * BQ_L
        for s in range(4):
            c = tv < pv[s]
            nv = jnp.where(c, tv, pv[s])
            ng = jnp.where(c, tg, pg[s])
            tv = jnp.where(c, pv[s], tv)
            tg = jnp.where(c, pg[s], tg)
            pv[s] = nv
            pg[s] = ng

    PV = jnp.concatenate(pv, axis=1)
    PG = jnp.concatenate(pg, axis=1)
    BIG = jnp.int32(1 << 30)
    cols = []
    m = None
    gi = None
    for _ in range(KNN):
        m = jnp.min(PV, axis=1, keepdims=True)
        eq = PV <= m
        gi = jnp.min(jnp.where(eq, PG, BIG), axis=1, keepdims=True)
        PV = jnp.where(eq & (PG == gi), INF, PV)
        cols.append(gi)
    pooled = jnp.concatenate(cols, axis=1)

    bad = (pv[3] < m) | ((pv[3] == m) & (pg[3] < gi))
    any_bad = jnp.sum(bad.astype(jnp.int32)) > 0

    def _full_extract(_):
        iota = lax.broadcasted_iota(jnp.int32, (BQ, N), 1)
        dd = d
        cs = []
        for _ in range(KNN):
            mm = jnp.min(dd, axis=1, keepdims=True)
            eq2 = dd <= mm
            ix = jnp.min(jnp.where(eq2, iota, N), axis=1, keepdims=True)
            dd = jnp.where(iota == ix, INF, dd)
            cs.append(ix)
        return jnp.concatenate(cs, axis=1)

    out = lax.cond(any_bad, _full_extract, lambda _: pooled, 0)
    out_ref[0] = out + b * N


def _topk(pos_b, posT_b, b):
    grid = (NBLK,)
    return pl.pallas_call(
        functools.partial(_topk_body, b=b),
        grid=grid,
        in_specs=[
            pl.BlockSpec((1, BQ, 3), lambda i: (0, i, 0)),
            pl.BlockSpec((1, 8, N), lambda i: (0, 0, 0)),
        ],
        out_specs=pl.BlockSpec((1, BQ, KNN), lambda i: (0, i, 0)),
        out_shape=jax.ShapeDtypeStruct((1, N, KNN), jnp.int32),
    )(pos_b, posT_b)


_SC_CHUNK = 128


def _sc_gather(kv, p128, idx_flat):
    info = plsc.get_sparse_core_info()
    nw = info.num_cores * info.num_subcores
    nrows = idx_flat.shape[0]
    rows_per_w = nrows // nw
    n_chunks = rows_per_w // _SC_CHUNK
    mesh = plsc.VectorSubcoreMesh(core_axis_name="c", subcore_axis_name="s")

    @functools.partial(
        pl.kernel,
        out_type=[
            jax.ShapeDtypeStruct((nrows, D), jnp.int32),
            jax.ShapeDtypeStruct((nrows, D), jnp.float32),
        ],
        mesh=mesh,
        scratch_types=[
            pltpu.VMEM((_SC_CHUNK,), jnp.int32),
            pltpu.VMEM((_SC_CHUNK, D), jnp.int32),
            pltpu.VMEM((_SC_CHUNK, D), jnp.float32),
            pltpu.SemaphoreType.DMA,
            pltpu.SemaphoreType.DMA,
        ],
    )
    def k(kv_hbm, p_hbm, idx_hbm, gkv_hbm, gp_hbm, idx_v, rkv, rp, s1, s2):
        wid = lax.axis_index("s") * info.num_cores + lax.axis_index("c")
        base_w = wid * rows_per_w

        def body(c, carry):
            base = base_w + c * _SC_CHUNK
            pltpu.sync_copy(idx_hbm.at[pl.ds(base, _SC_CHUNK)], idx_v)
            cp1 = pltpu.async_copy(kv_hbm.at[idx_v], rkv, s1)
            cp2 = pltpu.async_copy(p_hbm.at[idx_v], rp, s2)
            cp1.wait()
            cp2.wait()
            pltpu.sync_copy(rkv, gkv_hbm.at[pl.ds(base, _SC_CHUNK)])
            pltpu.sync_copy(rp, gp_hbm.at[pl.ds(base, _SC_CHUNK)])
            return carry

        lax.fori_loop(0, n_chunks, body, 0)

    return k(kv, p128, idx_flat)


_ISQ = 1.0 / (128.0 ** 0.5)


def _attn_body(q_ref, x_ref, pq_ref, gkv_ref, gp_ref, p1_ref, pb1_ref,
               p2_ref, pb2_ref, a1_ref, ab1_ref, a2_ref, ab2_ref,
               wf_ref, bf_ref, out_ref):
    R = BQ * KNN
    w = gkv_ref[...]
    kf = lax.bitcast_convert_type(w & jnp.int32(-65536), jnp.float32)
    vf = lax.bitcast_convert_type(lax.shift_left(w, 16), jnp.float32)
    posn = gp_ref[:, :PPAD]
    pq = pq_ref[:, :PPAD]
    pq_rep = jnp.broadcast_to(pq[:, None, :], (BQ, KNN, PPAD)).reshape(R, PPAD)
    rel = pq_rep - posn
    pe = jnp.dot(rel, p1_ref[...], preferred_element_type=jnp.float32)
    pe = jnp.maximum(pe + pb1_ref[...][None, :], 0.0)
    pe = jnp.dot(pe, p2_ref[...], preferred_element_type=jnp.float32)
    pe = pe + pb2_ref[...][None, :]

    q = q_ref[...]
    q_rep = jnp.broadcast_to(q[:, None, :], (BQ, KNN, D)).reshape(R, D)
    h = q_rep - kf + pe
    a = jnp.dot(h, a1_ref[...], preferred_element_type=jnp.float32)
    a = jnp.maximum(a + ab1_ref[...][None, :], 0.0)
    a = jnp.dot(a, a2_ref[...], preferred_element_type=jnp.float32)
    a = (a + ab2_ref[...][None, :]) * _ISQ

    a3 = a.reshape(BQ, KNN, D)
    m = jnp.max(a3, axis=1, keepdims=True)
    e = jnp.exp(a3 - m)
    s = jnp.sum(e, axis=1, keepdims=True)
    p = e / s
    v3 = (vf + pe).reshape(BQ, KNN, D)
    agg = jnp.sum(p * v3, axis=1)
    out = jnp.dot(agg, wf_ref[...], preferred_element_type=jnp.float32)
    out_ref[...] = out + bf_ref[...][None, :] + x_ref[...]


def _attn(q, xf, pq, gkv, gp, P1p, pb1, P2, pb2, A1, ab1, A2, ab2, Wf, bf):
    grid = (q.shape[0] // BQ,)
    R = BQ * KNN
    full = lambda shape: pl.BlockSpec(shape, lambda i: tuple(0 for _ in shape))
    return pl.pallas_call(
        _attn_body,
        grid=grid,
        in_specs=[
            pl.BlockSpec((BQ, D), lambda i: (i, 0)),
            pl.BlockSpec((BQ, D), lambda i: (i, 0)),
            pl.BlockSpec((BQ, D), lambda i: (i, 0)),
            pl.BlockSpec((R, D), lambda i: (i, 0)),
            pl.BlockSpec((R, D), lambda i: (i, 0)),
            full((PPAD, D)), full((D,)), full((D, D)), full((D,)),
            full((D, D)), full((D,)), full((D, D)), full((D,)),
            full((D, D)), full((D,)),
        ],
        out_specs=pl.BlockSpec((BQ, D), lambda i: (i, 0)),
        out_shape=jax.ShapeDtypeStruct((q.shape[0], D), jnp.float32),
    )(q, xf, pq, gkv, gp, P1p, pb1, P2, pb2, A1, ab1, A2, ab2, Wf, bf)


def kernel(x, pos, Wq, Wk, Wv, P1, pb1, P2, pb2, A1, ab1, A2, ab2, Wf, bf):
    xf = x.reshape(B * N, D)
    posf = pos.reshape(B * N, 3)
    q, kv, p128 = _prep(xf, posf, Wq, Wk, Wv)

    posT = jnp.swapaxes(pos, 1, 2)
    posT8 = jnp.concatenate(
        [posT, jnp.zeros((B, 5, N), jnp.float32)], axis=1)

    P1p = jnp.concatenate([P1, jnp.zeros((PPAD - 3, D), jnp.float32)], axis=0)

    knn = [None] * B
    gs = [None] * B
    outs = [None] * B
    for b in range(B):
        knn[b] = _topk(pos[b:b + 1], posT8[b:b + 1], b)
        gs[b] = _sc_gather(kv, p128, knn[b].reshape(N * KNN))
    for b in range(B):
        sl = slice(b * N, (b + 1) * N)
        outs[b] = _attn(q[sl], xf[sl], p128[sl], gs[b][0], gs[b][1],
                        P1p, pb1, P2, pb2, A1, ab1, A2, ab2, Wf, bf)
    out = jnp.concatenate(outs, axis=0)
    return out.reshape(B, N, D)
